# baseline (speedup 1.0000x reference)
import dataclasses
import functools

import jax
import jax.numpy as jnp
from jax import lax
from jax.experimental import pallas as pl
from jax.experimental.pallas import tpu as pltpu
from jax.experimental.pallas import tpu_sc as plsc

B, S = 4096, 200
N = B * S
COORD = 128
NC, NS = 2, 16
NW = NC * NS
W = 64
CHUNKS = N // (NW * W)
RPC = 6 * W // 128

OFF_L, OFF_R, OFF_W, OFF_T, OFF_D, OFF_H = 0, 512, 1024, 1536, 2048, 2560
TAB_ROWS = 6 * 512


def _compiler_params():
  cp = pltpu.CompilerParams()
  if "needs_layout_passes" in pltpu.CompilerParams.__dataclass_fields__:
    cp = dataclasses.replace(cp, needs_layout_passes=False)
  return cp


def _sc_gather(bbox_t, table):
  mesh = plsc.VectorSubcoreMesh(core_axis_name="c", subcore_axis_name="s")

  @functools.partial(
      pl.kernel,
      compiler_params=_compiler_params(),
      out_type=jax.ShapeDtypeStruct((6 * N // 128, 128, COORD), jnp.float32),
      mesh=mesh,
      scratch_types=[
          pltpu.VMEM_SHARED((TAB_ROWS, COORD), jnp.float32),
          pltpu.VMEM((4, 2 * W), jnp.int32),
          pltpu.VMEM((2, RPC, 128), jnp.int32),
          pltpu.VMEM((2, RPC, 128, COORD), jnp.float32),
          pltpu.SemaphoreType.DMA,
          pltpu.SemaphoreType.DMA,
          pltpu.SemaphoreType.DMA,
          pltpu.SemaphoreType.DMA,
          pltpu.SemaphoreType.DMA,
      ],
  )
  def k(bbox_hbm, tab_hbm, out_hbm, tab_s, bb_v, idx_v, rows_v,
        tsem, gsem0, gsem1, osem0, osem1):
    wid = lax.axis_index("s") * NC + lax.axis_index("c")
    base0 = wid * (CHUNKS * W)
    gsems = (gsem0, gsem1)
    osems = (osem0, osem1)

    @pl.when(lax.axis_index("s") == 0)
    def _():
      pltpu.async_copy(tab_hbm, tab_s, tsem).wait()

    plsc.subcore_barrier()

    iota6 = lax.iota(jnp.int32, 16) * 6

    def fetch_bbox(c):
      base = base0 + c * W
      pltpu.sync_copy(bbox_hbm.at[:, pl.ds(base, 2 * W)], bb_v)

    def prep(b):
      @pl.loop(0, W, step=16)
      def _(j):
        s = pl.ds(b * W + j, 16)
        x0 = bb_v[0, s]
        y0 = bb_v[1, s]
        x1 = bb_v[2, s]
        y1 = bb_v[3, s]
        vals = (
            x0 + OFF_L,
            x1 + OFF_R,
            y0 + OFF_T,
            y1 + OFF_D,
            (x1 - x0) + OFF_W,
            (y1 - y0) + OFF_H,
        )
        for t in range(6):
          pos = iota6 + (6 * j + t)
          plsc.store_scatter(
              idx_v.at[b],
              [lax.shift_right_logical(pos, 7), lax.bitwise_and(pos, 127)],
              vals[t])

    def start_gather(b):
      for r in range(RPC):
        pltpu.async_copy(tab_s.at[idx_v.at[b, r]], rows_v.at[b, r], gsems[b])

    def wait_gather(b):
      for r in range(RPC):
        pltpu.make_async_copy(
            tab_s.at[idx_v.at[b, r]], rows_v.at[b, r], gsems[b]).wait()

    def start_out(c, b):
      blk = (wid * CHUNKS + c) * RPC
      pltpu.async_copy(rows_v.at[b], out_hbm.at[pl.ds(blk, RPC)], osems[b])

    def wait_out(c, b):
      blk = (wid * CHUNKS + c) * RPC
      pltpu.make_async_copy(
          rows_v.at[b], out_hbm.at[pl.ds(blk, RPC)], osems[b]).wait()

    def step(c, b):
      if b == 0:
        fetch_bbox(c)
      wait_out(c - 2, b)
      prep(b)
      start_gather(b)
      wait_gather(1 - b)
      start_out(c - 1, 1 - b)

    fetch_bbox(0)
    prep(0)
    start_gather(0)
    prep(1)
    start_gather(1)
    wait_gather(0)
    start_out(0, 0)

    @pl.loop(1, CHUNKS // 2)
    def _(i):
      c = 2 * i
      step(c, 0)
      step(c + 1, 1)

    wait_gather(1)
    start_out(CHUNKS - 1, 1)
    wait_out(CHUNKS - 2, 0)
    wait_out(CHUNKS - 1, 1)

  return k(bbox_t, table)


def kernel(bbox, l_table, r_table, w_table, t_table, d_table, h_table):
  bbox_t = bbox.reshape(N, 4).astype(jnp.int32).T
  table = jnp.concatenate(
      [l_table[:512], r_table[:512], w_table[:512],
       t_table[:512], d_table[:512], h_table[:512]], axis=0)
  out = _sc_gather(bbox_t, table)
  return out.reshape(B, S, 6 * COORD)


# speedup vs baseline: 1.0005x; 1.0005x over previous
import dataclasses
import functools

import jax
import jax.numpy as jnp
from jax import lax
from jax.experimental import pallas as pl
from jax.experimental.pallas import tpu as pltpu
from jax.experimental.pallas import tpu_sc as plsc

B, S = 4096, 200
N = B * S
COORD = 128
NC, NS = 2, 16
NW = NC * NS
W = 64
CHUNKS = N // (NW * W)
RPC = 6 * W // 128

OFF_L, OFF_R, OFF_W, OFF_T, OFF_D, OFF_H = 0, 512, 1024, 1536, 2048, 2560
TAB_ROWS = 6 * 512


def _compiler_params():
  cp = pltpu.CompilerParams()
  if "needs_layout_passes" in pltpu.CompilerParams.__dataclass_fields__:
    cp = dataclasses.replace(cp, needs_layout_passes=False)
  return cp


def _sc_gather(bbox_t, table):
  mesh = plsc.VectorSubcoreMesh(core_axis_name="c", subcore_axis_name="s")

  @functools.partial(
      pl.kernel,
      compiler_params=_compiler_params(),
      out_type=jax.ShapeDtypeStruct((6 * N, COORD), jnp.float32),
      mesh=mesh,
      scratch_types=[
          pltpu.VMEM_SHARED((TAB_ROWS, COORD), jnp.float32),
          pltpu.VMEM((4, 2 * W), jnp.int32),
          pltpu.VMEM((2, RPC, 128), jnp.int32),
          pltpu.VMEM((2 * RPC, 128), jnp.int32),
          pltpu.VMEM((2, RPC, 128, COORD), jnp.float32),
          pltpu.SemaphoreType.DMA,
          pltpu.SemaphoreType.DMA,
          pltpu.SemaphoreType.DMA,
          pltpu.SemaphoreType.DMA,
          pltpu.SemaphoreType.DMA,
      ],
  )
  def k(bbox_hbm, tab_hbm, out_hbm, tab_s, bb_v, idx_v, oidx_v, rows_v,
        tsem, gsem0, gsem1, osem0, osem1):
    wid = lax.axis_index("s") * NC + lax.axis_index("c")
    base0 = wid * (CHUNKS * W)
    gsems = (gsem0, gsem1)
    osems = (osem0, osem1)

    @pl.when(lax.axis_index("s") == 0)
    def _():
      pltpu.async_copy(tab_hbm, tab_s, tsem).wait()

    plsc.subcore_barrier()

    iota6 = lax.iota(jnp.int32, 16) * 6
    iota16 = lax.iota(jnp.int32, 16)

    def fetch_bbox(c):
      base = base0 + c * W
      pltpu.sync_copy(bbox_hbm.at[:, pl.ds(base, 2 * W)], bb_v)

    def prep(b):
      @pl.loop(0, W, step=16)
      def _(j):
        s = pl.ds(b * W + j, 16)
        x0 = bb_v[0, s]
        y0 = bb_v[1, s]
        x1 = bb_v[2, s]
        y1 = bb_v[3, s]
        vals = (
            x0 + OFF_L,
            x1 + OFF_R,
            y0 + OFF_T,
            y1 + OFF_D,
            (x1 - x0) + OFF_W,
            (y1 - y0) + OFF_H,
        )
        for t in range(6):
          pos = iota6 + (6 * j + t)
          plsc.store_scatter(
              idx_v.at[b],
              [lax.shift_right_logical(pos, 7), lax.bitwise_and(pos, 127)],
              vals[t])

    def start_gather(b):
      for r in range(RPC):
        pltpu.async_copy(tab_s.at[idx_v.at[b, r]], rows_v.at[b, r], gsems[b])

    def wait_gather(b):
      for r in range(RPC):
        pltpu.make_async_copy(
            tab_s.at[idx_v.at[b, r]], rows_v.at[b, r], gsems[b]).wait()

    def start_out(c, b):
      row0 = (wid * CHUNKS + c) * (6 * W)
      for r in range(RPC):
        @pl.loop(0, 128, step=16)
        def _(j):
          oidx_v[b * RPC + r, pl.ds(j, 16)] = iota16 + (row0 + r * 128 + j)
        pltpu.async_copy(rows_v.at[b, r], out_hbm.at[oidx_v.at[b * RPC + r]],
                         osems[b])

    def wait_out(c, b):
      for r in range(RPC):
        pltpu.make_async_copy(
            rows_v.at[b, r], out_hbm.at[oidx_v.at[b * RPC + r]], osems[b]).wait()

    def step(c, b):
      if b == 0:
        fetch_bbox(c)
      wait_out(c - 2, b)
      prep(b)
      start_gather(b)
      wait_gather(1 - b)
      start_out(c - 1, 1 - b)

    fetch_bbox(0)
    prep(0)
    start_gather(0)
    prep(1)
    start_gather(1)
    wait_gather(0)
    start_out(0, 0)

    @pl.loop(1, CHUNKS // 2)
    def _(i):
      c = 2 * i
      step(c, 0)
      step(c + 1, 1)

    wait_gather(1)
    start_out(CHUNKS - 1, 1)
    wait_out(CHUNKS - 2, 0)
    wait_out(CHUNKS - 1, 1)

  return k(bbox_t, table)


def kernel(bbox, l_table, r_table, w_table, t_table, d_table, h_table):
  bbox_t = bbox.reshape(N, 4).astype(jnp.int32).T
  table = jnp.concatenate(
      [l_table[:512], r_table[:512], w_table[:512],
       t_table[:512], d_table[:512], h_table[:512]], axis=0)
  out = _sc_gather(bbox_t, table)
  return out.reshape(B, S, 6 * COORD)
